# baseline (device time: 37747 ns/iter reference)
import jax
import jax.numpy as jnp
from jax import lax
from jax.experimental import pallas as pl
from jax.experimental.pallas import tpu as pltpu

B, S, D, DC_SHARD = 2, 256, 1024, 64
H, DH, DR = 16, 64, 32
BS = B * S


def kernel(x, Wdkv, Wuk, Wuv, Wq, Wqr, Wkr, Wo):
    def body(x_hbm, wdkv_ref, wuk_hbm, wuv_hbm, wq_hbm, wqr_hbm, wkr_ref,
             wo_hbm, out_hbm, x_v, wuk_v, wuv_v, wq_v, wqr_v, wo_v, out_v,
             c_self, c_peer, wuk_peer, wuv_peer, o_acc,
             load_sems, send_sems, recv_sems, out_sem):
        my_x = lax.axis_index("x")
        my_y = lax.axis_index("y")
        peer = (1 - my_x, my_y)

        ld_x = pltpu.make_async_copy(x_hbm, x_v, load_sems.at[0])
        ld_x.start()
        ld_wq = pltpu.make_async_copy(wq_hbm, wq_v, load_sems.at[1])
        ld_wq.start()
        ld_wqr = pltpu.make_async_copy(wqr_hbm, wqr_v, load_sems.at[2])
        ld_wqr.start()
        ld_wo = pltpu.make_async_copy(wo_hbm, wo_v, load_sems.at[3])
        ld_wo.start()
        ld_wuk = pltpu.make_async_copy(wuk_hbm, wuk_v, load_sems.at[4])
        ld_wuk.start()
        ld_wuv = pltpu.make_async_copy(wuv_hbm, wuv_v, load_sems.at[5])
        ld_wuv.start()

        barrier_sem = pltpu.get_barrier_semaphore()
        pl.semaphore_signal(barrier_sem, inc=1, device_id=peer,
                            device_id_type=pl.DeviceIdType.MESH)
        pl.semaphore_wait(barrier_sem, 1)

        ld_wuk.wait()
        rdma_wuk = pltpu.make_async_remote_copy(
            src_ref=wuk_v, dst_ref=wuk_peer,
            send_sem=send_sems.at[0], recv_sem=recv_sems.at[0],
            device_id=peer, device_id_type=pl.DeviceIdType.MESH)
        rdma_wuk.start()
        ld_wuv.wait()
        rdma_wuv = pltpu.make_async_remote_copy(
            src_ref=wuv_v, dst_ref=wuv_peer,
            send_sem=send_sems.at[1], recv_sem=recv_sems.at[1],
            device_id=peer, device_id_type=pl.DeviceIdType.MESH)
        rdma_wuv.start()

        ld_x.wait()
        x2d = x_v[...].reshape(BS, D)
        c_self[...] = jnp.dot(x2d, wdkv_ref[...],
                              preferred_element_type=jnp.float32)
        rdma_c = pltpu.make_async_remote_copy(
            src_ref=c_self, dst_ref=c_peer,
            send_sem=send_sems.at[2], recv_sem=recv_sems.at[2],
            device_id=peer, device_id_type=pl.DeviceIdType.MESH)
        rdma_c.start()

        ld_wq.wait()
        q = jnp.dot(x2d, wq_v[...], preferred_element_type=jnp.float32)
        ld_wqr.wait()
        qr = jnp.dot(x2d, wqr_v[...], preferred_element_type=jnp.float32)
        kr = jnp.dot(x2d, wkr_ref[...], preferred_element_type=jnp.float32)

        rdma_wuk.wait()
        rdma_wuv.wait()
        rdma_c.wait()

        c_mine = c_self[...]
        c_oth = c_peer[...]
        k = (jnp.dot(c_mine, wuk_v[...], preferred_element_type=jnp.float32)
             + jnp.dot(c_oth, wuk_peer[...],
                       preferred_element_type=jnp.float32))
        v = (jnp.dot(c_mine, wuv_v[...], preferred_element_type=jnp.float32)
             + jnp.dot(c_oth, wuv_peer[...],
                       preferred_element_type=jnp.float32))

        scale = (DH + DR) ** -0.5
        for b in range(B):
            r0 = b * S
            kr_b = kr[r0:r0 + S, :]
            for h in range(H):
                c0 = h * DH
                qb = q[r0:r0 + S, c0:c0 + DH]
                qrb = qr[r0:r0 + S, h * DR:(h + 1) * DR]
                kb = k[r0:r0 + S, c0:c0 + DH]
                vb = v[r0:r0 + S, c0:c0 + DH]
                s = (lax.dot_general(qb, kb, (((1,), (1,)), ((), ())),
                                     preferred_element_type=jnp.float32)
                     + lax.dot_general(qrb, kr_b, (((1,), (1,)), ((), ())),
                                       preferred_element_type=jnp.float32))
                p = jnp.exp(s * scale)
                denom = jnp.sum(p, axis=-1, keepdims=True)
                o = jnp.dot(p, vb, preferred_element_type=jnp.float32)
                o_acc[r0:r0 + S, c0:c0 + DH] = o / denom

        ld_wo.wait()
        out2d = jnp.dot(o_acc[...], wo_v[...],
                        preferred_element_type=jnp.float32)
        out_v[...] = out2d.reshape(B, S, D)
        st_out = pltpu.make_async_copy(out_v, out_hbm, out_sem)
        st_out.start()
        st_out.wait()

    vmem = pl.BlockSpec(memory_space=pltpu.VMEM)
    hbm = pl.BlockSpec(memory_space=pltpu.HBM)
    return pl.pallas_call(
        body,
        out_shape=jax.ShapeDtypeStruct((B, S, D), jnp.float32),
        in_specs=[hbm, vmem, hbm, hbm, hbm, hbm, vmem, hbm],
        out_specs=hbm,
        scratch_shapes=[
            pltpu.VMEM((B, S, D), jnp.float32),
            pltpu.VMEM((DC_SHARD, D), jnp.float32),
            pltpu.VMEM((DC_SHARD, D), jnp.float32),
            pltpu.VMEM((D, D), jnp.float32),
            pltpu.VMEM((D, H * DR), jnp.float32),
            pltpu.VMEM((D, D), jnp.float32),
            pltpu.VMEM((B, S, D), jnp.float32),
            pltpu.VMEM((BS, DC_SHARD), jnp.float32),
            pltpu.VMEM((BS, DC_SHARD), jnp.float32),
            pltpu.VMEM((DC_SHARD, D), jnp.float32),
            pltpu.VMEM((DC_SHARD, D), jnp.float32),
            pltpu.VMEM((BS, D), jnp.float32),
            pltpu.SemaphoreType.DMA((6,)),
            pltpu.SemaphoreType.DMA((3,)),
            pltpu.SemaphoreType.DMA((3,)),
            pltpu.SemaphoreType.DMA,
        ],
        compiler_params=pltpu.CompilerParams(collective_id=0),
    )(x, Wdkv, Wuk, Wuv, Wq, Wqr, Wkr, Wo)


# device time: 31956 ns/iter; 1.1812x vs baseline; 1.1812x over previous
import jax
import jax.numpy as jnp
from jax import lax
from jax.experimental import pallas as pl
from jax.experimental.pallas import tpu as pltpu

B, S, D, DC_SHARD = 2, 256, 1024, 64
H, DH, DR = 16, 64, 32
BS = B * S


def kernel(x, Wdkv, Wuk, Wuv, Wq, Wqr, Wkr, Wo):
    hbm_space = pltpu.MemorySpace.HBM
    x = pltpu.with_memory_space_constraint(x, hbm_space)
    Wuk = pltpu.with_memory_space_constraint(Wuk, hbm_space)
    Wuv = pltpu.with_memory_space_constraint(Wuv, hbm_space)
    Wq = pltpu.with_memory_space_constraint(Wq, hbm_space)
    Wqr = pltpu.with_memory_space_constraint(Wqr, hbm_space)
    Wo = pltpu.with_memory_space_constraint(Wo, hbm_space)

    def body(x_hbm, wdkv_ref, wuk_hbm, wuv_hbm, wq_hbm, wqr_hbm, wkr_ref,
             wo_hbm, out_hbm, x_v, wuk_v, wuv_v, wq_v, wqr_v, wo_v, out_v,
             c_self, c_peer, wuk_peer, wuv_peer, o_acc,
             load_sems, send_sems, recv_sems, out_sem):
        my_x = lax.axis_index("x")
        my_y = lax.axis_index("y")
        peer = (1 - my_x, my_y)

        ld_x = pltpu.make_async_copy(x_hbm, x_v, load_sems.at[0])
        ld_x.start()
        ld_wq = pltpu.make_async_copy(wq_hbm, wq_v, load_sems.at[1])
        ld_wq.start()
        ld_wqr = pltpu.make_async_copy(wqr_hbm, wqr_v, load_sems.at[2])
        ld_wqr.start()
        ld_wo = pltpu.make_async_copy(wo_hbm, wo_v, load_sems.at[3])
        ld_wo.start()
        ld_wuk = pltpu.make_async_copy(wuk_hbm, wuk_v, load_sems.at[4])
        ld_wuk.start()
        ld_wuv = pltpu.make_async_copy(wuv_hbm, wuv_v, load_sems.at[5])
        ld_wuv.start()

        barrier_sem = pltpu.get_barrier_semaphore()
        pl.semaphore_signal(barrier_sem, inc=1, device_id=peer,
                            device_id_type=pl.DeviceIdType.MESH)
        pl.semaphore_wait(barrier_sem, 1)

        ld_wuk.wait()
        rdma_wuk = pltpu.make_async_remote_copy(
            src_ref=wuk_v, dst_ref=wuk_peer,
            send_sem=send_sems.at[0], recv_sem=recv_sems.at[0],
            device_id=peer, device_id_type=pl.DeviceIdType.MESH)
        rdma_wuk.start()
        ld_wuv.wait()
        rdma_wuv = pltpu.make_async_remote_copy(
            src_ref=wuv_v, dst_ref=wuv_peer,
            send_sem=send_sems.at[1], recv_sem=recv_sems.at[1],
            device_id=peer, device_id_type=pl.DeviceIdType.MESH)
        rdma_wuv.start()

        ld_x.wait()
        x2d = x_v[...].reshape(BS, D)
        c_self[...] = jnp.dot(x2d, wdkv_ref[...],
                              preferred_element_type=jnp.float32)
        rdma_c = pltpu.make_async_remote_copy(
            src_ref=c_self, dst_ref=c_peer,
            send_sem=send_sems.at[2], recv_sem=recv_sems.at[2],
            device_id=peer, device_id_type=pl.DeviceIdType.MESH)
        rdma_c.start()

        ld_wq.wait()
        q = jnp.dot(x2d, wq_v[...], preferred_element_type=jnp.float32)
        ld_wqr.wait()
        qr = jnp.dot(x2d, wqr_v[...], preferred_element_type=jnp.float32)
        kr = jnp.dot(x2d, wkr_ref[...], preferred_element_type=jnp.float32)

        rdma_wuk.wait()
        rdma_wuv.wait()
        rdma_c.wait()

        c_mine = c_self[...]
        c_oth = c_peer[...]
        k = (jnp.dot(c_mine, wuk_v[...], preferred_element_type=jnp.float32)
             + jnp.dot(c_oth, wuk_peer[...],
                       preferred_element_type=jnp.float32))
        v = (jnp.dot(c_mine, wuv_v[...], preferred_element_type=jnp.float32)
             + jnp.dot(c_oth, wuv_peer[...],
                       preferred_element_type=jnp.float32))

        scale = (DH + DR) ** -0.5
        for b in range(B):
            r0 = b * S
            kr_b = kr[r0:r0 + S, :]
            for h in range(H):
                c0 = h * DH
                qb = q[r0:r0 + S, c0:c0 + DH]
                qrb = qr[r0:r0 + S, h * DR:(h + 1) * DR]
                kb = k[r0:r0 + S, c0:c0 + DH]
                vb = v[r0:r0 + S, c0:c0 + DH]
                s = (lax.dot_general(qb, kb, (((1,), (1,)), ((), ())),
                                     preferred_element_type=jnp.float32)
                     + lax.dot_general(qrb, kr_b, (((1,), (1,)), ((), ())),
                                       preferred_element_type=jnp.float32))
                p = jnp.exp(s * scale)
                denom = jnp.sum(p, axis=-1, keepdims=True)
                o = jnp.dot(p, vb, preferred_element_type=jnp.float32)
                o_acc[r0:r0 + S, c0:c0 + DH] = o / denom

        ld_wo.wait()
        out2d = jnp.dot(o_acc[...], wo_v[...],
                        preferred_element_type=jnp.float32)
        out_v[...] = out2d.reshape(B, S, D)
        st_out = pltpu.make_async_copy(out_v, out_hbm, out_sem)
        st_out.start()
        st_out.wait()

    vmem = pl.BlockSpec(memory_space=pltpu.VMEM)
    hbm = pl.BlockSpec(memory_space=pltpu.HBM)
    return pl.pallas_call(
        body,
        out_shape=jax.ShapeDtypeStruct((B, S, D), jnp.float32),
        in_specs=[hbm, vmem, hbm, hbm, hbm, hbm, vmem, hbm],
        out_specs=hbm,
        scratch_shapes=[
            pltpu.VMEM((B, S, D), jnp.float32),
            pltpu.VMEM((DC_SHARD, D), jnp.float32),
            pltpu.VMEM((DC_SHARD, D), jnp.float32),
            pltpu.VMEM((D, D), jnp.float32),
            pltpu.VMEM((D, H * DR), jnp.float32),
            pltpu.VMEM((D, D), jnp.float32),
            pltpu.VMEM((B, S, D), jnp.float32),
            pltpu.VMEM((BS, DC_SHARD), jnp.float32),
            pltpu.VMEM((BS, DC_SHARD), jnp.float32),
            pltpu.VMEM((DC_SHARD, D), jnp.float32),
            pltpu.VMEM((DC_SHARD, D), jnp.float32),
            pltpu.VMEM((BS, D), jnp.float32),
            pltpu.SemaphoreType.DMA((6,)),
            pltpu.SemaphoreType.DMA((3,)),
            pltpu.SemaphoreType.DMA((3,)),
            pltpu.SemaphoreType.DMA,
        ],
        compiler_params=pltpu.CompilerParams(collective_id=0),
    )(x, Wdkv, Wuk, Wuv, Wq, Wqr, Wkr, Wo)
